# baseline (speedup 1.0000x reference)
import functools

import jax
import jax.numpy as jnp
from jax import lax
from jax.experimental import pallas as pl
from jax.experimental.pallas import tpu as pltpu
from jax.experimental.pallas import tpu_sc as plsc

B, N, D, H, L = 2, 1024, 128, 8, 2
DH = D // H
T, FF = 17, 512
TP = 32
R = 512
NB = N // R
SCALE = 1.0 / (DH ** 0.5)


def _embed(table, idx_flat):
    info = plsc.get_sparse_core_info()
    nw = info.num_cores * info.num_subcores
    nb = idx_flat.shape[0]
    bw = nb // nw
    mesh = plsc.VectorSubcoreMesh(core_axis_name="c", subcore_axis_name="s")

    @functools.partial(
        pl.kernel,
        mesh=mesh,
        out_type=jax.ShapeDtypeStruct((nb, D), jnp.float32),
        scratch_types=[
            pltpu.VMEM((bw,), jnp.int32),
            pltpu.VMEM((bw, D), jnp.float32),
            pltpu.SemaphoreType.DMA,
        ],
    )
    def k(table_hbm, idx_hbm, out_hbm, idx_v, rows_v, sem):
        wid = lax.axis_index("s") * info.num_cores + lax.axis_index("c")
        base = wid * bw
        pltpu.sync_copy(idx_hbm.at[pl.ds(base, bw)], idx_v)
        pltpu.async_copy(table_hbm.at[idx_v], rows_v, sem).wait()
        pltpu.sync_copy(rows_v, out_hbm.at[pl.ds(base, bw)])

    return k(table, idx_flat)


def _gat_body(x_ref, pos_ref, et_ref, etab_ref, wq_ref, wk_ref,
              wv_ref, wo_ref, w1_ref, b1_ref, w2_ref, b2_ref, g1_ref,
              be1_ref, g2_ref, be2_ref, out_ref, xc_s, k_s, v_s, ek_s, o_s):
    li = pl.program_id(0)
    b = pl.program_id(1)
    i = pl.program_id(2)
    bf = jnp.bfloat16

    @pl.when((li == 0) & (i == 0))
    def _init_x():
        xc_s[b] = x_ref[0] + pos_ref[0]

    @pl.when(i == 0)
    def _prep():
        xb = xc_s[b].astype(bf)
        k_s[...] = jnp.dot(xb, wk_ref[0].astype(bf),
                           preferred_element_type=jnp.float32).astype(bf)
        v_s[...] = jnp.dot(xb, wv_ref[0].astype(bf),
                           preferred_element_type=jnp.float32).astype(bf)
        ek_s[...] = jnp.dot(etab_ref[...].astype(bf), wk_ref[0].astype(bf),
                            preferred_element_type=jnp.float32).astype(bf)

    x_blk = xc_s[b, pl.ds(i * R, R), :]
    q = jnp.dot(x_blk.astype(bf), wq_ref[0].astype(bf),
                preferred_element_type=jnp.float32)
    qb = (q * SCALE).astype(bf)
    et_v = et_ref[0]

    sc = [None] * H
    qe = [None] * H
    for h in range(H):
        hs = pl.ds(h * DH, DH)
        q_h = qb[:, h * DH:(h + 1) * DH]
        sc[h] = lax.dot_general(q_h, k_s[:, hs], (((1,), (1,)), ((), ())),
                                preferred_element_type=jnp.float32).astype(bf)
        qe[h] = lax.dot_general(q_h, ek_s[:, hs], (((1,), (1,)), ((), ())),
                                preferred_element_type=jnp.float32).astype(bf)
    bits = [(et_v & jnp.int16(1 << k)) != 0 for k in range(5)]
    amask = (et_v & jnp.int16(32)) != 0
    for h in range(H):
        lvl = [lax.slice(qe[h], (0, t), (R, t + 1)) for t in range(T)]
        for k in range(5):
            nxt = [jnp.where(bits[k], lvl[2 * p + 1], lvl[2 * p])
                   for p in range(len(lvl) // 2)]
            if len(lvl) % 2:
                nxt.append(lvl[-1])
            lvl = nxt
        s = jnp.where(amask, -1e9, (sc[h] + lvl[0]).astype(jnp.float32))
        m = jnp.max(s, axis=-1, keepdims=True)
        p = jnp.exp(s - m)
        den = jnp.sum(p, axis=-1, keepdims=True)
        o_s[:, pl.ds(h * DH, DH)] = jnp.dot(
            p.astype(bf), v_s[:, pl.ds(h * DH, DH)],
            preferred_element_type=jnp.float32) * (1.0 / den)

    att = jnp.dot(o_s[...].astype(bf), wo_ref[0].astype(bf),
                  preferred_element_type=jnp.float32)
    x1 = x_blk + att
    mu = jnp.mean(x1, axis=-1, keepdims=True)
    dx = x1 - mu
    var = jnp.mean(dx * dx, axis=-1, keepdims=True)
    x1 = dx / jnp.sqrt(var + 1e-5) * g1_ref[0, 0] + be1_ref[0, 0]
    ff = jnp.maximum(
        jnp.dot(x1.astype(bf), w1_ref[0].astype(bf),
                preferred_element_type=jnp.float32)
        + b1_ref[0, 0], 0.0)
    ff = jnp.dot(ff.astype(bf), w2_ref[0].astype(bf),
                 preferred_element_type=jnp.float32) + b2_ref[0, 0]
    x2 = x1 + ff
    mu = jnp.mean(x2, axis=-1, keepdims=True)
    dx = x2 - mu
    var = jnp.mean(dx * dx, axis=-1, keepdims=True)
    x2 = dx / jnp.sqrt(var + 1e-5) * g2_ref[0, 0] + be2_ref[0, 0]
    xc_s[b, pl.ds(i * R, R), :] = x2
    out_ref[0] = x2


def _gat(x, pos, et, etab, Wq, Wk, Wv, Wo, W1, b1, W2, b2,
         g1, be1, g2, be2):
    full3 = lambda l, b, i: (b, 0, 0)
    rows3 = lambda l, b, i: (b, i, 0)
    wl = lambda l, b, i: (l, 0, 0)
    fixed = lambda l, b, i: (0, 0)
    in_specs = [
        pl.BlockSpec((1, N, D), full3),
        pl.BlockSpec((1, N, D), lambda l, b, i: (0, 0, 0)),
        pl.BlockSpec((1, R, N), rows3),
        pl.BlockSpec((TP, D), fixed),
        pl.BlockSpec((1, D, D), wl),
        pl.BlockSpec((1, D, D), wl),
        pl.BlockSpec((1, D, D), wl),
        pl.BlockSpec((1, D, D), wl),
        pl.BlockSpec((1, D, FF), wl),
        pl.BlockSpec((1, 1, FF), wl),
        pl.BlockSpec((1, FF, D), wl),
        pl.BlockSpec((1, 1, D), wl),
        pl.BlockSpec((1, 1, D), wl),
        pl.BlockSpec((1, 1, D), wl),
        pl.BlockSpec((1, 1, D), wl),
        pl.BlockSpec((1, 1, D), wl),
    ]
    return pl.pallas_call(
        _gat_body,
        grid=(L, B, NB),
        in_specs=in_specs,
        out_specs=pl.BlockSpec((1, R, D), rows3),
        out_shape=jax.ShapeDtypeStruct((B, N, D), jnp.float32),
        scratch_shapes=[
            pltpu.VMEM((B, N, D), jnp.float32),
            pltpu.VMEM((N, D), jnp.bfloat16),
            pltpu.VMEM((N, D), jnp.bfloat16),
            pltpu.VMEM((TP, D), jnp.bfloat16),
            pltpu.VMEM((R, D), jnp.float32),
        ],
    )(x, pos, et, etab, Wq, Wk, Wv, Wo, W1, b1, W2, b2,
      g1, be1, g2, be2)


def kernel(word_ids, position_ids, adj, edge_types, token_table, pos_table,
           edge_table, Wq, Wk, Wv, Wo, W1, b1, W2, b2, g1, be1, g2, be2):
    et = jnp.where(adj, edge_types, 32).astype(jnp.int16)
    etab = jnp.pad(edge_table, ((0, TP - T), (0, 0)))
    tok = _embed(token_table, word_ids.reshape(-1).astype(jnp.int32))
    x = tok.reshape(B, N, D)
    pos = pos_table.reshape(1, N, D)
    return _gat(x, pos, et, etab, Wq, Wk, Wv, Wo,
                W1, b1.reshape(L, 1, FF), W2, b2.reshape(L, 1, D),
                g1.reshape(L, 1, D), be1.reshape(L, 1, D),
                g2.reshape(L, 1, D), be2.reshape(L, 1, D))


# speedup vs baseline: 1.1095x; 1.1095x over previous
import functools

import jax
import jax.numpy as jnp
from jax import lax
from jax.experimental import pallas as pl
from jax.experimental.pallas import tpu as pltpu
from jax.experimental.pallas import tpu_sc as plsc

B, N, D, H, L = 2, 1024, 128, 8, 2
DH = D // H
T, FF = 17, 512
TP = 32
R = 512
NB = N // R
SCALE = 1.0 / (DH ** 0.5)


def _embed(table, idx_flat):
    info = plsc.get_sparse_core_info()
    nw = info.num_cores * info.num_subcores
    nb = idx_flat.shape[0]
    bw = nb // nw
    mesh = plsc.VectorSubcoreMesh(core_axis_name="c", subcore_axis_name="s")

    @functools.partial(
        pl.kernel,
        mesh=mesh,
        out_type=jax.ShapeDtypeStruct((nb, D), jnp.float32),
        scratch_types=[
            pltpu.VMEM((bw,), jnp.int32),
            pltpu.VMEM((bw, D), jnp.float32),
            pltpu.SemaphoreType.DMA,
        ],
    )
    def k(table_hbm, idx_hbm, out_hbm, idx_v, rows_v, sem):
        wid = lax.axis_index("s") * info.num_cores + lax.axis_index("c")
        base = wid * bw
        pltpu.sync_copy(idx_hbm.at[pl.ds(base, bw)], idx_v)
        pltpu.async_copy(table_hbm.at[idx_v], rows_v, sem).wait()
        pltpu.sync_copy(rows_v, out_hbm.at[pl.ds(base, bw)])

    return k(table, idx_flat)


def _gat_body(x_ref, pos_ref, et_ref, etab_ref, wq_ref, wk_ref,
              wv_ref, wo_ref, w1_ref, b1_ref, w2_ref, b2_ref, g1_ref,
              be1_ref, g2_ref, be2_ref, out_ref, xc_s, k_s, v_s, ek_s, o_s):
    li = pl.program_id(0)
    b = pl.program_id(1)
    i = pl.program_id(2)
    bf = jnp.bfloat16

    @pl.when((li == 0) & (i == 0))
    def _init_x():
        xc_s[b] = x_ref[0] + pos_ref[0]

    @pl.when(i == 0)
    def _prep():
        xb = xc_s[b].astype(bf)
        k_s[...] = jnp.dot(xb, wk_ref[0].astype(bf),
                           preferred_element_type=jnp.float32).astype(bf)
        v_s[...] = jnp.dot(xb, wv_ref[0].astype(bf),
                           preferred_element_type=jnp.float32).astype(bf)
        ek_s[...] = jnp.dot(etab_ref[...].astype(bf), wk_ref[0].astype(bf),
                            preferred_element_type=jnp.float32).astype(bf)

    x_blk = xc_s[b, pl.ds(i * R, R), :]
    q = jnp.dot(x_blk.astype(bf), wq_ref[0].astype(bf),
                preferred_element_type=jnp.float32)
    qb = (q * SCALE).astype(bf)
    et_v = et_ref[0]

    sc = [None] * H
    qe = [None] * H
    for h in range(H):
        hs = pl.ds(h * DH, DH)
        q_h = qb[:, h * DH:(h + 1) * DH]
        sc[h] = lax.dot_general(q_h, k_s[:, hs], (((1,), (1,)), ((), ())),
                                preferred_element_type=jnp.float32).astype(bf)
        qe[h] = lax.dot_general(q_h, ek_s[:, hs], (((1,), (1,)), ((), ())),
                                preferred_element_type=jnp.float32).astype(bf)
    bits = [(et_v & jnp.int16(1 << k)) != 0 for k in range(5)]
    amask = (et_v & jnp.int16(32)) != 0
    for h in range(H):
        lvl = [lax.slice(qe[h], (0, t), (R, t + 1)) for t in range(T)]
        for k in range(5):
            nxt = [jnp.where(bits[k], lvl[2 * p + 1], lvl[2 * p])
                   for p in range(len(lvl) // 2)]
            if len(lvl) % 2:
                nxt.append(lvl[-1])
            lvl = nxt
        sc[h] = sc[h] + lvl[0]
    for h in range(H):
        hs = pl.ds(h * DH, DH)
        s = jnp.where(amask, -1e9, sc[h].astype(jnp.float32))
        m = jnp.max(s, axis=-1, keepdims=True)
        p = jnp.exp(s - m)
        den = jnp.sum(p, axis=-1, keepdims=True)
        o_s[:, hs] = jnp.dot(p.astype(bf), v_s[:, hs],
                             preferred_element_type=jnp.float32) * (1.0 / den)

    att = jnp.dot(o_s[...].astype(bf), wo_ref[0].astype(bf),
                  preferred_element_type=jnp.float32)
    x1 = x_blk + att
    mu = jnp.mean(x1, axis=-1, keepdims=True)
    dx = x1 - mu
    var = jnp.mean(dx * dx, axis=-1, keepdims=True)
    x1 = dx / jnp.sqrt(var + 1e-5) * g1_ref[0, 0] + be1_ref[0, 0]
    ff = jnp.maximum(
        jnp.dot(x1.astype(bf), w1_ref[0].astype(bf),
                preferred_element_type=jnp.float32)
        + b1_ref[0, 0], 0.0)
    ff = jnp.dot(ff.astype(bf), w2_ref[0].astype(bf),
                 preferred_element_type=jnp.float32) + b2_ref[0, 0]
    x2 = x1 + ff
    mu = jnp.mean(x2, axis=-1, keepdims=True)
    dx = x2 - mu
    var = jnp.mean(dx * dx, axis=-1, keepdims=True)
    x2 = dx / jnp.sqrt(var + 1e-5) * g2_ref[0, 0] + be2_ref[0, 0]
    xc_s[b, pl.ds(i * R, R), :] = x2
    out_ref[0] = x2


def _gat(x, pos, et, etab, Wq, Wk, Wv, Wo, W1, b1, W2, b2,
         g1, be1, g2, be2):
    full3 = lambda l, b, i: (b, 0, 0)
    rows3 = lambda l, b, i: (b, i, 0)
    wl = lambda l, b, i: (l, 0, 0)
    fixed = lambda l, b, i: (0, 0)
    in_specs = [
        pl.BlockSpec((1, N, D), full3),
        pl.BlockSpec((1, N, D), lambda l, b, i: (0, 0, 0)),
        pl.BlockSpec((1, R, N), rows3),
        pl.BlockSpec((TP, D), fixed),
        pl.BlockSpec((1, D, D), wl),
        pl.BlockSpec((1, D, D), wl),
        pl.BlockSpec((1, D, D), wl),
        pl.BlockSpec((1, D, D), wl),
        pl.BlockSpec((1, D, FF), wl),
        pl.BlockSpec((1, 1, FF), wl),
        pl.BlockSpec((1, FF, D), wl),
        pl.BlockSpec((1, 1, D), wl),
        pl.BlockSpec((1, 1, D), wl),
        pl.BlockSpec((1, 1, D), wl),
        pl.BlockSpec((1, 1, D), wl),
        pl.BlockSpec((1, 1, D), wl),
    ]
    return pl.pallas_call(
        _gat_body,
        grid=(L, B, NB),
        in_specs=in_specs,
        out_specs=pl.BlockSpec((1, R, D), rows3),
        out_shape=jax.ShapeDtypeStruct((B, N, D), jnp.float32),
        scratch_shapes=[
            pltpu.VMEM((B, N, D), jnp.float32),
            pltpu.VMEM((N, D), jnp.bfloat16),
            pltpu.VMEM((N, D), jnp.bfloat16),
            pltpu.VMEM((TP, D), jnp.bfloat16),
            pltpu.VMEM((R, D), jnp.float32),
        ],
    )(x, pos, et, etab, Wq, Wk, Wv, Wo, W1, b1, W2, b2,
      g1, be1, g2, be2)


def kernel(word_ids, position_ids, adj, edge_types, token_table, pos_table,
           edge_table, Wq, Wk, Wv, Wo, W1, b1, W2, b2, g1, be1, g2, be2):
    et = jnp.where(adj, edge_types, 32).astype(jnp.int16)
    etab = jnp.pad(edge_table, ((0, TP - T), (0, 0)))
    tok = _embed(token_table, word_ids.reshape(-1).astype(jnp.int32))
    x = tok.reshape(B, N, D)
    pos = pos_table.reshape(1, N, D)
    return _gat(x, pos, et, etab, Wq, Wk, Wv, Wo,
                W1, b1.reshape(L, 1, FF), W2, b2.reshape(L, 1, D),
                g1.reshape(L, 1, D), be1.reshape(L, 1, D),
                g2.reshape(L, 1, D), be2.reshape(L, 1, D))
